# baseline (speedup 1.0000x reference)
import functools
import jax
import jax.numpy as jnp
from jax import lax
from jax.experimental import pallas as pl
from jax.experimental.pallas import tpu as pltpu

LEAK = 0.1
BN_EPS = 1e-5
TILE_E = 4096
HALF_E = 512
TILE_P = 16384
GROUP = 4
BF = jnp.bfloat16


def _round_up(a, b):
    return (a + b - 1) // b * b


def _dot(a, b):
    return jnp.dot(a, b, preferred_element_type=jnp.float32)


def _dot_tb(a, b):
    return lax.dot_general(a, b, (((1,), (1,)), ((), ())),
                           preferred_element_type=jnp.float32)


def _dot_ta_tb(a, b):
    return lax.dot_general(a, b, (((0,), (1,)), ((), ())),
                           preferred_element_type=jnp.float32)


def _leaky(h):
    return jnp.maximum(h, h * LEAK)


def _eye(n):
    r = lax.broadcasted_iota(jnp.int32, (n, n), 0)
    c = lax.broadcasted_iota(jnp.int32, (n, n), 1)
    return jnp.where(r == c, 1.0, 0.0)


def _outer(row, width):
    ones = jnp.ones((1, width), jnp.float32)
    return lax.dot_general(row, ones, (((0,), (0,)), ((), ())),
                           preferred_element_type=jnp.float32)


def _stats_kernel(e_ref, out_ref):
    e = e_ref[...]
    s = jnp.sum(e, axis=0, keepdims=True)
    ss = jnp.sum(e * e, axis=0, keepdims=True)
    out_ref[...] = jnp.concatenate([s, ss], axis=0).reshape(1, 2, -1)


def _edge_stats(e_pad, n_ef, n_tp):
    return pl.pallas_call(
        _stats_kernel,
        grid=(n_tp,),
        in_specs=[pl.BlockSpec((TILE_P, n_ef), lambda t: (t, 0))],
        out_specs=pl.BlockSpec((1, 2, n_ef), lambda t: (t, 0, 0)),
        out_shape=jax.ShapeDtypeStruct((n_tp, 2, n_ef), jnp.float32),
        compiler_params=pltpu.CompilerParams(
            dimension_semantics=("parallel",)),
    )(e_pad)


def _table_kernel(part_ref, xg_ref, gx_ref, bx_ref, ge_ref, be_ref,
                  w1_ref, b1_ref, w2_ref, b2_ref, w3_ref, b3_ref,
                  w4_ref, b4_ref, w5_ref, b5_ref, w6_ref, b6_ref,
                  agt_ref, bgt_ref, w1ee_ref, b1cb_ref,
                  w2t_ref, b2cb_ref, w3t_ref, b3cb_ref,
                  w4t_ref, b4cb_ref, w5t_ref, b5cb_ref,
                  w6t_ref, b6cb_ref,
                  *, n_edges, n_nf):
    w1e = w1_ref[2 * n_nf:, :]
    s = jnp.sum(part_ref[...], axis=0)
    inv_n = jnp.float32(1.0 / n_edges)
    mean_e = s[0:1, :] * inv_n
    var_e = s[1:2, :] * inv_n - mean_e * mean_e
    scale_e = ge_ref[...] * lax.rsqrt(var_e + BN_EPS)
    shift_e = be_ref[...] - mean_e * scale_e
    w1et = _dot_ta_tb(w1e, _eye(n_nf))
    w1ee_ref[...] = (w1et * scale_e).astype(BF)
    b1_eff = b1_ref[...] + _dot_tb(shift_e, w1et)
    b1cb_ref[...] = _outer(b1_eff, HALF_E)

    xg = xg_ref[...]
    s4 = jnp.mean(xg, axis=0, keepdims=True)
    ss4 = jnp.mean(xg * xg, axis=0, keepdims=True)
    m = jnp.zeros((1, n_nf), jnp.float32)
    msq = jnp.zeros((1, n_nf), jnp.float32)
    for k in range(GROUP):
        m = m + s4[:, k * n_nf:(k + 1) * n_nf]
        msq = msq + ss4[:, k * n_nf:(k + 1) * n_nf]
    m = m * (1.0 / GROUP)
    msq = msq * (1.0 / GROUP)
    var_n = msq - m * m
    scale_n = gx_ref[...] * lax.rsqrt(var_n + BN_EPS)
    shift_n = bx_ref[...] - m * scale_n
    scale4 = jnp.concatenate([scale_n] * GROUP, axis=1)
    shift4 = jnp.concatenate([shift_n] * GROUP, axis=1)
    xn = xg * scale4 + shift4
    w1s = w1_ref[0:n_nf, :]
    w1d = w1_ref[n_nf:2 * n_nf, :]
    for k in range(GROUP):
        xk = xn[:, k * n_nf:(k + 1) * n_nf]
        agt_ref[k * 64:(k + 1) * 64, :] = _dot_ta_tb(w1s, xk).astype(BF)
        bgt_ref[k * 64:(k + 1) * 64, :] = _dot_ta_tb(w1d, xk).astype(BF)

    for w_ref, b_ref, wt_ref, bcb_ref in (
            (w2_ref, b2_ref, w2t_ref, b2cb_ref),
            (w3_ref, b3_ref, w3t_ref, b3cb_ref),
            (w4_ref, b4_ref, w4t_ref, b4cb_ref),
            (w5_ref, b5_ref, w5t_ref, b5cb_ref)):
        w = w_ref[...]
        wt_ref[...] = _dot_tb(_eye(w.shape[1]), w).astype(BF)
        bcb_ref[...] = _outer(b_ref[...], HALF_E)
    w6 = w6_ref[...]
    w6t = _dot_tb(_eye(2), w6)
    w6t_ref[...] = jnp.concatenate(
        [w6t, jnp.zeros((6, 8), jnp.float32)], axis=0).astype(BF)
    b6cb_ref[...] = _outer(
        jnp.concatenate([b6_ref[...], jnp.zeros((1, 6), jnp.float32)],
                        axis=1), HALF_E)


def _build_tables(partials, xg, gx, bx, ge, be, ws, n_edges):
    n_groups = xg.shape[0]
    n_nf = gx.shape[1]
    n_ef = ws[0].shape[0] - 2 * n_nf
    args = [partials, xg, gx, bx, ge, be] + list(ws)
    outs = [
        ((GROUP * 64, n_groups), BF),
        ((GROUP * 64, n_groups), BF),
        ((64, n_ef), BF),
        ((64, HALF_E), jnp.float32),
        ((64, 64), BF), ((64, HALF_E), jnp.float32),
        ((32, 64), BF), ((32, HALF_E), jnp.float32),
        ((16, 32), BF), ((16, HALF_E), jnp.float32),
        ((8, 16), BF), ((8, HALF_E), jnp.float32),
        ((8, 8), BF), ((8, HALF_E), jnp.float32),
    ]
    return pl.pallas_call(
        functools.partial(_table_kernel, n_edges=n_edges, n_nf=n_nf),
        grid=(1,),
        in_specs=[pl.BlockSpec(a.shape, lambda t, n=len(a.shape): (0,) * n)
                  for a in args],
        out_specs=[pl.BlockSpec(s, lambda t: (0, 0)) for s, _ in outs],
        out_shape=[jax.ShapeDtypeStruct(s, d) for s, d in outs],
        compiler_params=pltpu.CompilerParams(
            dimension_semantics=("arbitrary",)),
    )(*args)


def _main_kernel(agt_ref, bgt_ref, w1ee_ref, b1cb_ref,
                 w2t_ref, b2cb_ref, w3t_ref, b3cb_ref,
                 w4t_ref, b4cb_ref, w5t_ref, b5cb_ref,
                 w6t_ref, b6cb_ref,
                 e_ref, idx_ref, out_ref, *, n_groups):
    agt = agt_ref[...]
    bgt = bgt_ref[...]
    w1ee = w1ee_ref[...]
    b1cb = b1cb_ref[...]
    iota_g = lax.broadcasted_iota(jnp.int32, (n_groups, HALF_E), 0)
    n_half = TILE_E // HALF_E
    slices = [slice(h * HALF_E, (h + 1) * HALF_E) for h in range(n_half)]

    hs = []
    for sl in slices:
        src = idx_ref[0:1, sl]
        dst = idx_ref[1:2, sl]
        ms = jnp.where(iota_g == (src >> 2), 1.0, 0.0).astype(BF)
        md = jnp.where(iota_g == (dst >> 2), 1.0, 0.0).astype(BF)
        gs = _dot(agt, ms)
        gd = _dot(bgt, md)
        srcr = src & 3
        dstr = dst & 3
        h1 = b1cb + _dot_tb(w1ee, e_ref[sl, :].astype(BF))
        for k in range(GROUP):
            fs = jnp.where(srcr == k, 1.0, 0.0)
            fd = jnp.where(dstr == k, 1.0, 0.0)
            h1 = h1 + fs * gs[k * 64:(k + 1) * 64, :]
            h1 = h1 + fd * gd[k * 64:(k + 1) * 64, :]
        hs.append(_leaky(h1).astype(BF))
    for wt_ref, bcb_ref in ((w2t_ref, b2cb_ref), (w3t_ref, b3cb_ref),
                            (w4t_ref, b4cb_ref), (w5t_ref, b5cb_ref)):
        wt = wt_ref[...]
        bcb = bcb_ref[...]
        hs = [_leaky(_dot(wt, h) + bcb).astype(BF) for h in hs]
    w6t = w6t_ref[...]
    b6cb = b6cb_ref[...]
    for h, sl in zip(hs, slices):
        out8 = _dot(w6t, h) + b6cb
        out_ref[:, sl] = out8[0:2, :]


def _edge_mlp(tables, e_pad, idx_pad, n_tiles):
    n_groups = tables[0].shape[1]
    n_ef = e_pad.shape[1]
    e_rows = e_pad.shape[0]
    in_specs = (
        [pl.BlockSpec(a.shape, lambda t: (0, 0)) for a in tables]
        + [pl.BlockSpec((TILE_E, n_ef), lambda t: (t, 0)),
           pl.BlockSpec((2, TILE_E), lambda t: (0, t))]
    )
    mlp_flops = 2 * (64 * (2 * GROUP * 64 + n_ef) + 64 * 64 + 32 * 64
                     + 16 * 32 + 8 * 16 + 8 * 8)
    cost = pl.CostEstimate(
        flops=e_rows * mlp_flops + 2 * e_rows * n_groups * GROUP * 64,
        transcendentals=0,
        bytes_accessed=4 * e_rows * (n_ef + 4),
    )
    return pl.pallas_call(
        functools.partial(_main_kernel, n_groups=n_groups),
        grid=(n_tiles,),
        in_specs=in_specs,
        out_specs=pl.BlockSpec((2, TILE_E), lambda t: (0, t)),
        out_shape=jax.ShapeDtypeStruct((2, e_rows), jnp.float32),
        compiler_params=pltpu.CompilerParams(
            dimension_semantics=("parallel",)),
        cost_estimate=cost,
    )(*tables, e_pad, idx_pad)


def kernel(x, edge_index, e, xbatch,
           bn_node_gamma, bn_node_beta, bn_edge_gamma, bn_edge_beta,
           w1, b1, w2, b2, w3, b3, w4, b4, w5, b5, w6, b6):
    del xbatch
    f32 = jnp.float32
    n_nodes, n_nf = x.shape
    n_edges, n_ef = e.shape
    n_groups = n_nodes // GROUP

    e_rows = _round_up(max(n_edges, 1), max(TILE_E, TILE_P))
    e_pad = jnp.pad(e.astype(f32), ((0, e_rows - n_edges), (0, 0)))
    idx_pad = jnp.pad(edge_index.astype(jnp.int32),
                      ((0, 0), (0, e_rows - n_edges)))

    xg = x.astype(f32).reshape(n_groups, GROUP * n_nf)

    partials = _edge_stats(e_pad, n_ef, e_rows // TILE_P)
    ws = [w1.astype(f32), b1.reshape(1, -1).astype(f32),
          w2.astype(f32), b2.reshape(1, -1).astype(f32),
          w3.astype(f32), b3.reshape(1, -1).astype(f32),
          w4.astype(f32), b4.reshape(1, -1).astype(f32),
          w5.astype(f32), b5.reshape(1, -1).astype(f32),
          w6.astype(f32), b6.reshape(1, -1).astype(f32)]
    tables = _build_tables(
        partials, xg,
        bn_node_gamma.reshape(1, -1).astype(f32),
        bn_node_beta.reshape(1, -1).astype(f32),
        bn_edge_gamma.reshape(1, -1).astype(f32),
        bn_edge_beta.reshape(1, -1).astype(f32),
        ws, n_edges)
    out2 = _edge_mlp(tables, e_pad, idx_pad, e_rows // TILE_E)
    return {'edge_pred': [out2[:, :n_edges].T]}


# speedup vs baseline: 1.0250x; 1.0250x over previous
import functools
import jax
import jax.numpy as jnp
from jax import lax
from jax.experimental import pallas as pl
from jax.experimental.pallas import tpu as pltpu

LEAK = 0.1
BN_EPS = 1e-5
TILE_E = 16384
HALF_E = 1024
TILE_P = 16384
GROUP = 4
BF = jnp.bfloat16


def _round_up(a, b):
    return (a + b - 1) // b * b


def _dot(a, b):
    return jnp.dot(a, b, preferred_element_type=jnp.float32)


def _dot_tb(a, b):
    return lax.dot_general(a, b, (((1,), (1,)), ((), ())),
                           preferred_element_type=jnp.float32)


def _dot_ta_tb(a, b):
    return lax.dot_general(a, b, (((0,), (1,)), ((), ())),
                           preferred_element_type=jnp.float32)


def _leaky(h):
    return jnp.maximum(h, h * LEAK)


def _eye(n):
    r = lax.broadcasted_iota(jnp.int32, (n, n), 0)
    c = lax.broadcasted_iota(jnp.int32, (n, n), 1)
    return jnp.where(r == c, 1.0, 0.0)


def _outer(row, width):
    ones = jnp.ones((1, width), jnp.float32)
    return lax.dot_general(row, ones, (((0,), (0,)), ((), ())),
                           preferred_element_type=jnp.float32)


def _stats_kernel(e_ref, out_ref):
    e = e_ref[...]
    s = jnp.sum(e, axis=0, keepdims=True)
    ss = jnp.sum(e * e, axis=0, keepdims=True)
    out_ref[...] = jnp.concatenate([s, ss], axis=0).reshape(1, 2, -1)


def _edge_stats(e_pad, n_ef, n_tp):
    return pl.pallas_call(
        _stats_kernel,
        grid=(n_tp,),
        in_specs=[pl.BlockSpec((TILE_P, n_ef), lambda t: (t, 0))],
        out_specs=pl.BlockSpec((1, 2, n_ef), lambda t: (t, 0, 0)),
        out_shape=jax.ShapeDtypeStruct((n_tp, 2, n_ef), jnp.float32),
        compiler_params=pltpu.CompilerParams(
            dimension_semantics=("parallel",)),
    )(e_pad)


def _table_kernel(part_ref, xg_ref, gx_ref, bx_ref, ge_ref, be_ref,
                  w1_ref, b1_ref, w2_ref, b2_ref, w3_ref, b3_ref,
                  w4_ref, b4_ref, w5_ref, b5_ref, w6_ref, b6_ref,
                  agt_ref, bgt_ref, w1ee_ref, b1cb_ref,
                  w2t_ref, b2cb_ref, w3t_ref, b3cb_ref,
                  w4t_ref, b4cb_ref, w5t_ref, b5cb_ref,
                  w6t_ref, b6cb_ref,
                  *, n_edges, n_nf):
    w1e = w1_ref[2 * n_nf:, :]
    s = jnp.sum(part_ref[...], axis=0)
    inv_n = jnp.float32(1.0 / n_edges)
    mean_e = s[0:1, :] * inv_n
    var_e = s[1:2, :] * inv_n - mean_e * mean_e
    scale_e = ge_ref[...] * lax.rsqrt(var_e + BN_EPS)
    shift_e = be_ref[...] - mean_e * scale_e
    w1et = _dot_ta_tb(w1e, _eye(n_nf))
    w1ee_ref[...] = (w1et * scale_e).astype(BF)
    b1_eff = b1_ref[...] + _dot_tb(shift_e, w1et)
    b1cb_ref[...] = _outer(b1_eff, HALF_E)

    xg = xg_ref[...]
    s4 = jnp.mean(xg, axis=0, keepdims=True)
    ss4 = jnp.mean(xg * xg, axis=0, keepdims=True)
    m = jnp.zeros((1, n_nf), jnp.float32)
    msq = jnp.zeros((1, n_nf), jnp.float32)
    for k in range(GROUP):
        m = m + s4[:, k * n_nf:(k + 1) * n_nf]
        msq = msq + ss4[:, k * n_nf:(k + 1) * n_nf]
    m = m * (1.0 / GROUP)
    msq = msq * (1.0 / GROUP)
    var_n = msq - m * m
    scale_n = gx_ref[...] * lax.rsqrt(var_n + BN_EPS)
    shift_n = bx_ref[...] - m * scale_n
    scale4 = jnp.concatenate([scale_n] * GROUP, axis=1)
    shift4 = jnp.concatenate([shift_n] * GROUP, axis=1)
    xn = xg * scale4 + shift4
    w1s = w1_ref[0:n_nf, :]
    w1d = w1_ref[n_nf:2 * n_nf, :]
    for k in range(GROUP):
        xk = xn[:, k * n_nf:(k + 1) * n_nf]
        agt_ref[k * 64:(k + 1) * 64, :] = _dot_ta_tb(w1s, xk).astype(BF)
        bgt_ref[k * 64:(k + 1) * 64, :] = _dot_ta_tb(w1d, xk).astype(BF)

    for w_ref, b_ref, wt_ref, bcb_ref in (
            (w2_ref, b2_ref, w2t_ref, b2cb_ref),
            (w3_ref, b3_ref, w3t_ref, b3cb_ref),
            (w4_ref, b4_ref, w4t_ref, b4cb_ref),
            (w5_ref, b5_ref, w5t_ref, b5cb_ref)):
        w = w_ref[...]
        wt_ref[...] = _dot_tb(_eye(w.shape[1]), w).astype(BF)
        bcb_ref[...] = _outer(b_ref[...], HALF_E)
    w6 = w6_ref[...]
    w6t = _dot_tb(_eye(2), w6)
    w6t_ref[...] = jnp.concatenate(
        [w6t, jnp.zeros((6, 8), jnp.float32)], axis=0).astype(BF)
    b6cb_ref[...] = _outer(
        jnp.concatenate([b6_ref[...], jnp.zeros((1, 6), jnp.float32)],
                        axis=1), HALF_E)


def _build_tables(partials, xg, gx, bx, ge, be, ws, n_edges):
    n_groups = xg.shape[0]
    n_nf = gx.shape[1]
    n_ef = ws[0].shape[0] - 2 * n_nf
    args = [partials, xg, gx, bx, ge, be] + list(ws)
    outs = [
        ((GROUP * 64, n_groups), BF),
        ((GROUP * 64, n_groups), BF),
        ((64, n_ef), BF),
        ((64, HALF_E), jnp.float32),
        ((64, 64), BF), ((64, HALF_E), jnp.float32),
        ((32, 64), BF), ((32, HALF_E), jnp.float32),
        ((16, 32), BF), ((16, HALF_E), jnp.float32),
        ((8, 16), BF), ((8, HALF_E), jnp.float32),
        ((8, 8), BF), ((8, HALF_E), jnp.float32),
    ]
    return pl.pallas_call(
        functools.partial(_table_kernel, n_edges=n_edges, n_nf=n_nf),
        grid=(1,),
        in_specs=[pl.BlockSpec(a.shape, lambda t, n=len(a.shape): (0,) * n)
                  for a in args],
        out_specs=[pl.BlockSpec(s, lambda t: (0, 0)) for s, _ in outs],
        out_shape=[jax.ShapeDtypeStruct(s, d) for s, d in outs],
        compiler_params=pltpu.CompilerParams(
            dimension_semantics=("arbitrary",)),
    )(*args)


def _main_kernel(agt_ref, bgt_ref, w1ee_ref, b1cb_ref,
                 w2t_ref, b2cb_ref, w3t_ref, b3cb_ref,
                 w4t_ref, b4cb_ref, w5t_ref, b5cb_ref,
                 w6t_ref, b6cb_ref,
                 e_ref, idx_ref, out_ref, *, n_groups):
    agt = agt_ref[...]
    bgt = bgt_ref[...]
    w1ee = w1ee_ref[...]
    b1cb = b1cb_ref[...]
    iota_g = lax.broadcasted_iota(jnp.int32, (n_groups, HALF_E), 0)
    n_half = TILE_E // HALF_E
    slices = [slice(h * HALF_E, (h + 1) * HALF_E) for h in range(n_half)]

    hs = []
    for sl in slices:
        src = idx_ref[0:1, sl]
        dst = idx_ref[1:2, sl]
        ms = jnp.where(iota_g == (src >> 2), 1.0, 0.0).astype(BF)
        md = jnp.where(iota_g == (dst >> 2), 1.0, 0.0).astype(BF)
        gs = _dot(agt, ms)
        gd = _dot(bgt, md)
        srcr = src & 3
        dstr = dst & 3
        h1 = b1cb + _dot_tb(w1ee, e_ref[sl, :].astype(BF))
        for k in range(GROUP):
            fs = jnp.where(srcr == k, 1.0, 0.0)
            fd = jnp.where(dstr == k, 1.0, 0.0)
            h1 = h1 + fs * gs[k * 64:(k + 1) * 64, :]
            h1 = h1 + fd * gd[k * 64:(k + 1) * 64, :]
        hs.append(_leaky(h1).astype(BF))
    for wt_ref, bcb_ref in ((w2t_ref, b2cb_ref), (w3t_ref, b3cb_ref),
                            (w4t_ref, b4cb_ref), (w5t_ref, b5cb_ref)):
        wt = wt_ref[...]
        bcb = bcb_ref[...]
        hs = [_leaky(_dot(wt, h) + bcb).astype(BF) for h in hs]
    w6t = w6t_ref[...]
    b6cb = b6cb_ref[...]
    for h, sl in zip(hs, slices):
        out8 = _dot(w6t, h) + b6cb
        out_ref[:, sl] = out8[0:2, :]


def _edge_mlp(tables, e_pad, idx_pad, n_tiles):
    n_groups = tables[0].shape[1]
    n_ef = e_pad.shape[1]
    e_rows = e_pad.shape[0]
    in_specs = (
        [pl.BlockSpec(a.shape, lambda t: (0, 0)) for a in tables]
        + [pl.BlockSpec((TILE_E, n_ef), lambda t: (t, 0)),
           pl.BlockSpec((2, TILE_E), lambda t: (0, t))]
    )
    mlp_flops = 2 * (64 * (2 * GROUP * 64 + n_ef) + 64 * 64 + 32 * 64
                     + 16 * 32 + 8 * 16 + 8 * 8)
    cost = pl.CostEstimate(
        flops=e_rows * mlp_flops + 2 * e_rows * n_groups * GROUP * 64,
        transcendentals=0,
        bytes_accessed=4 * e_rows * (n_ef + 4),
    )
    return pl.pallas_call(
        functools.partial(_main_kernel, n_groups=n_groups),
        grid=(n_tiles,),
        in_specs=in_specs,
        out_specs=pl.BlockSpec((2, TILE_E), lambda t: (0, t)),
        out_shape=jax.ShapeDtypeStruct((2, e_rows), jnp.float32),
        compiler_params=pltpu.CompilerParams(
            dimension_semantics=("parallel",)),
        cost_estimate=cost,
    )(*tables, e_pad, idx_pad)


def kernel(x, edge_index, e, xbatch,
           bn_node_gamma, bn_node_beta, bn_edge_gamma, bn_edge_beta,
           w1, b1, w2, b2, w3, b3, w4, b4, w5, b5, w6, b6):
    del xbatch
    f32 = jnp.float32
    n_nodes, n_nf = x.shape
    n_edges, n_ef = e.shape
    n_groups = n_nodes // GROUP

    e_rows = _round_up(max(n_edges, 1), max(TILE_E, TILE_P))
    e_pad = jnp.pad(e.astype(f32), ((0, e_rows - n_edges), (0, 0)))
    idx_pad = jnp.pad(edge_index.astype(jnp.int32),
                      ((0, 0), (0, e_rows - n_edges)))

    xg = x.astype(f32).reshape(n_groups, GROUP * n_nf)

    partials = _edge_stats(e_pad, n_ef, e_rows // TILE_P)
    ws = [w1.astype(f32), b1.reshape(1, -1).astype(f32),
          w2.astype(f32), b2.reshape(1, -1).astype(f32),
          w3.astype(f32), b3.reshape(1, -1).astype(f32),
          w4.astype(f32), b4.reshape(1, -1).astype(f32),
          w5.astype(f32), b5.reshape(1, -1).astype(f32),
          w6.astype(f32), b6.reshape(1, -1).astype(f32)]
    tables = _build_tables(
        partials, xg,
        bn_node_gamma.reshape(1, -1).astype(f32),
        bn_node_beta.reshape(1, -1).astype(f32),
        bn_edge_gamma.reshape(1, -1).astype(f32),
        bn_edge_beta.reshape(1, -1).astype(f32),
        ws, n_edges)
    out2 = _edge_mlp(tables, e_pad, idx_pad, e_rows // TILE_E)
    return {'edge_pred': [out2[:, :n_edges].T]}


# speedup vs baseline: 1.0786x; 1.0523x over previous
import functools
import jax
import jax.numpy as jnp
from jax import lax
from jax.experimental import pallas as pl
from jax.experimental.pallas import tpu as pltpu

LEAK = 0.1
BN_EPS = 1e-5
TILE_E = 16384
HALF_E = 1024
GROUP = 4
BF = jnp.bfloat16


def _round_up(a, b):
    return (a + b - 1) // b * b


def _dot(a, b):
    return jnp.dot(a, b, preferred_element_type=jnp.float32)


def _dot_tb(a, b):
    return lax.dot_general(a, b, (((1,), (1,)), ((), ())),
                           preferred_element_type=jnp.float32)


def _dot_ta_tb(a, b):
    return lax.dot_general(a, b, (((0,), (1,)), ((), ())),
                           preferred_element_type=jnp.float32)


def _leaky(h):
    return jnp.maximum(h, h * LEAK)


def _eye(n):
    r = lax.broadcasted_iota(jnp.int32, (n, n), 0)
    c = lax.broadcasted_iota(jnp.int32, (n, n), 1)
    return jnp.where(r == c, 1.0, 0.0)


def _outer(row, width):
    ones = jnp.ones((1, width), jnp.float32)
    return lax.dot_general(row, ones, (((0,), (0,)), ((), ())),
                           preferred_element_type=jnp.float32)


def _fused_kernel(e_hbm, idx_ref, xg_ref, gx_ref, bx_ref, ge_ref, be_ref,
                  w1_ref, b1_ref, w2_ref, b2_ref, w3_ref, b3_ref,
                  w4_ref, b4_ref, w5_ref, b5_ref, w6_ref, b6_ref,
                  out_ref,
                  e_scr, sums_ref, agt_ref, bgt_ref, w1ee_ref, b1cb_ref,
                  w2t_ref, b2cb_ref, w3t_ref, b3cb_ref,
                  w4t_ref, b4cb_ref, w5t_ref, b5cb_ref,
                  w6t_ref, b6cb_ref, sems,
                  *, n_edges, n_nf, n_groups, n_tiles):
    t = pl.program_id(0)

    @pl.when(t == 0)
    def _():
        sums_ref[...] = jnp.zeros_like(sums_ref)
        for i in range(n_tiles):
            pltpu.make_async_copy(
                e_hbm.at[pl.ds(i * TILE_E, TILE_E), :],
                e_scr.at[pl.ds(i * TILE_E, TILE_E), :],
                sems.at[i]).start()

    @pl.when(t < n_tiles)
    def _():
        off = pl.multiple_of(t * TILE_E, TILE_E)
        pltpu.make_async_copy(
            e_hbm.at[pl.ds(off, TILE_E), :],
            e_scr.at[pl.ds(off, TILE_E), :],
            sems.at[t]).wait()
        e = e_scr[pl.ds(off, TILE_E), :]
        sums_ref[0:1, :] = sums_ref[0:1, :] + jnp.sum(e, 0, keepdims=True)
        sums_ref[1:2, :] = sums_ref[1:2, :] + jnp.sum(e * e, 0, keepdims=True)

    @pl.when(t == n_tiles - 1)
    def _():
        w1e = w1_ref[2 * n_nf:, :]
        s = sums_ref[...]
        inv_n = jnp.float32(1.0 / n_edges)
        mean_e = s[0:1, :] * inv_n
        var_e = s[1:2, :] * inv_n - mean_e * mean_e
        scale_e = ge_ref[...] * lax.rsqrt(var_e + BN_EPS)
        shift_e = be_ref[...] - mean_e * scale_e
        w1et = _dot_ta_tb(w1e, _eye(n_nf))
        w1ee_ref[...] = (w1et * scale_e).astype(BF)
        b1_eff = b1_ref[...] + _dot_tb(shift_e, w1et)
        b1cb_ref[...] = _outer(b1_eff, HALF_E)

        xg = xg_ref[...]
        s4 = jnp.mean(xg, axis=0, keepdims=True)
        ss4 = jnp.mean(xg * xg, axis=0, keepdims=True)
        m = jnp.zeros((1, n_nf), jnp.float32)
        msq = jnp.zeros((1, n_nf), jnp.float32)
        for k in range(GROUP):
            m = m + s4[:, k * n_nf:(k + 1) * n_nf]
            msq = msq + ss4[:, k * n_nf:(k + 1) * n_nf]
        m = m * (1.0 / GROUP)
        msq = msq * (1.0 / GROUP)
        var_n = msq - m * m
        scale_n = gx_ref[...] * lax.rsqrt(var_n + BN_EPS)
        shift_n = bx_ref[...] - m * scale_n
        scale4 = jnp.concatenate([scale_n] * GROUP, axis=1)
        shift4 = jnp.concatenate([shift_n] * GROUP, axis=1)
        xn = xg * scale4 + shift4
        w1s = w1_ref[0:n_nf, :]
        w1d = w1_ref[n_nf:2 * n_nf, :]
        for k in range(GROUP):
            xk = xn[:, k * n_nf:(k + 1) * n_nf]
            agt_ref[k * 64:(k + 1) * 64, :] = _dot_ta_tb(w1s, xk).astype(BF)
            bgt_ref[k * 64:(k + 1) * 64, :] = _dot_ta_tb(w1d, xk).astype(BF)

        for w_ref, b_ref, wt_ref, bcb_ref in (
                (w2_ref, b2_ref, w2t_ref, b2cb_ref),
                (w3_ref, b3_ref, w3t_ref, b3cb_ref),
                (w4_ref, b4_ref, w4t_ref, b4cb_ref),
                (w5_ref, b5_ref, w5t_ref, b5cb_ref)):
            w = w_ref[...]
            wt_ref[...] = _dot_tb(_eye(w.shape[1]), w).astype(BF)
            bcb_ref[...] = _outer(b_ref[...], HALF_E)
        w6t = _dot_tb(_eye(2), w6_ref[...])
        w6t_ref[...] = jnp.concatenate(
            [w6t, jnp.zeros((6, 8), jnp.float32)], axis=0).astype(BF)
        b6cb_ref[...] = _outer(
            jnp.concatenate([b6_ref[...], jnp.zeros((1, 6), jnp.float32)],
                            axis=1), HALF_E)

    @pl.when(t >= n_tiles)
    def _():
        tb = t - n_tiles
        base = pl.multiple_of(tb * TILE_E, TILE_E)
        agt = agt_ref[...]
        bgt = bgt_ref[...]
        w1ee = w1ee_ref[...]
        b1cb = b1cb_ref[...]
        iota_g = lax.broadcasted_iota(jnp.int32, (n_groups, HALF_E), 0)
        n_half = TILE_E // HALF_E
        slices = [slice(h * HALF_E, (h + 1) * HALF_E) for h in range(n_half)]

        hs = []
        for h, sl in enumerate(slices):
            src = idx_ref[0:1, sl]
            dst = idx_ref[1:2, sl]
            ms = jnp.where(iota_g == (src >> 2), 1.0, 0.0).astype(BF)
            md = jnp.where(iota_g == (dst >> 2), 1.0, 0.0).astype(BF)
            gs = _dot(agt, ms)
            gd = _dot(bgt, md)
            srcr = src & 3
            dstr = dst & 3
            e_blk = e_scr[pl.ds(base + h * HALF_E, HALF_E), :]
            h1 = b1cb + _dot_tb(w1ee, e_blk.astype(BF))
            for k in range(GROUP):
                fs = jnp.where(srcr == k, 1.0, 0.0)
                fd = jnp.where(dstr == k, 1.0, 0.0)
                h1 = h1 + fs * gs[k * 64:(k + 1) * 64, :]
                h1 = h1 + fd * gd[k * 64:(k + 1) * 64, :]
            hs.append(_leaky(h1).astype(BF))
        for wt_ref, bcb_ref in ((w2t_ref, b2cb_ref), (w3t_ref, b3cb_ref),
                                (w4t_ref, b4cb_ref), (w5t_ref, b5cb_ref)):
            wt = wt_ref[...]
            bcb = bcb_ref[...]
            hs = [_leaky(_dot(wt, hh) + bcb).astype(BF) for hh in hs]
        w6t = w6t_ref[...]
        b6cb = b6cb_ref[...]
        for hh, sl in zip(hs, slices):
            out8 = _dot(w6t, hh) + b6cb
            out_ref[:, sl] = out8[0:2, :]


def kernel(x, edge_index, e, xbatch,
           bn_node_gamma, bn_node_beta, bn_edge_gamma, bn_edge_beta,
           w1, b1, w2, b2, w3, b3, w4, b4, w5, b5, w6, b6):
    del xbatch
    f32 = jnp.float32
    n_nodes, n_nf = x.shape
    n_edges, n_ef = e.shape
    n_groups = n_nodes // GROUP

    e_rows = _round_up(max(n_edges, 1), TILE_E)
    n_tiles = e_rows // TILE_E
    e_pad = jnp.pad(e.astype(f32), ((0, e_rows - n_edges), (0, 0)))
    idx_pad = jnp.pad(edge_index.astype(jnp.int32),
                      ((0, 0), (0, e_rows - n_edges)))
    xg = x.astype(f32).reshape(n_groups, GROUP * n_nf)

    smalls = [xg,
              bn_node_gamma.reshape(1, -1).astype(f32),
              bn_node_beta.reshape(1, -1).astype(f32),
              bn_edge_gamma.reshape(1, -1).astype(f32),
              bn_edge_beta.reshape(1, -1).astype(f32),
              w1.astype(f32), b1.reshape(1, -1).astype(f32),
              w2.astype(f32), b2.reshape(1, -1).astype(f32),
              w3.astype(f32), b3.reshape(1, -1).astype(f32),
              w4.astype(f32), b4.reshape(1, -1).astype(f32),
              w5.astype(f32), b5.reshape(1, -1).astype(f32),
              w6.astype(f32), b6.reshape(1, -1).astype(f32)]

    def _idx_map(t):
        return (0, jnp.maximum(t - n_tiles, 0))

    in_specs = (
        [pl.BlockSpec(memory_space=pl.ANY),
         pl.BlockSpec((2, TILE_E), _idx_map)]
        + [pl.BlockSpec(a.shape, lambda t, n=len(a.shape): (0,) * n)
           for a in smalls]
    )
    scratch = [
        pltpu.VMEM((e_rows, n_ef), f32),
        pltpu.VMEM((2, n_ef), f32),
        pltpu.VMEM((GROUP * 64, n_groups), BF),
        pltpu.VMEM((GROUP * 64, n_groups), BF),
        pltpu.VMEM((64, n_ef), BF),
        pltpu.VMEM((64, HALF_E), f32),
        pltpu.VMEM((64, 64), BF), pltpu.VMEM((64, HALF_E), f32),
        pltpu.VMEM((32, 64), BF), pltpu.VMEM((32, HALF_E), f32),
        pltpu.VMEM((16, 32), BF), pltpu.VMEM((16, HALF_E), f32),
        pltpu.VMEM((8, 16), BF), pltpu.VMEM((8, HALF_E), f32),
        pltpu.VMEM((8, 8), BF), pltpu.VMEM((8, HALF_E), f32),
        pltpu.SemaphoreType.DMA((n_tiles,)),
    ]
    mlp_flops = 2 * (64 * (2 * GROUP * 64 + n_ef) + 64 * 64 + 32 * 64
                     + 16 * 32 + 8 * 16 + 8 * 8)
    cost = pl.CostEstimate(
        flops=e_rows * mlp_flops + 2 * e_rows * n_groups * GROUP * 64,
        transcendentals=0,
        bytes_accessed=4 * e_rows * (n_ef + 4),
    )
    out2 = pl.pallas_call(
        functools.partial(_fused_kernel, n_edges=n_edges, n_nf=n_nf,
                          n_groups=n_groups, n_tiles=n_tiles),
        grid=(2 * n_tiles,),
        in_specs=in_specs,
        out_specs=pl.BlockSpec((2, TILE_E), _idx_map),
        out_shape=jax.ShapeDtypeStruct((2, e_rows), f32),
        scratch_shapes=scratch,
        compiler_params=pltpu.CompilerParams(
            dimension_semantics=("arbitrary",)),
        cost_estimate=cost,
    )(e_pad, idx_pad, *smalls)
    return {'edge_pred': [out2[:, :n_edges].T]}


# speedup vs baseline: 1.0880x; 1.0088x over previous
import functools
import jax
import jax.numpy as jnp
from jax import lax
from jax.experimental import pallas as pl
from jax.experimental.pallas import tpu as pltpu

LEAK = 0.1
BN_EPS = 1e-5
TILE_E = 16384
HALF_E = 1024
GROUP = 4
BF = jnp.bfloat16


def _round_up(a, b):
    return (a + b - 1) // b * b


def _dot(a, b):
    return jnp.dot(a, b, preferred_element_type=jnp.float32)


def _dot_tb(a, b):
    return lax.dot_general(a, b, (((1,), (1,)), ((), ())),
                           preferred_element_type=jnp.float32)


def _dot_ta_tb(a, b):
    return lax.dot_general(a, b, (((0,), (1,)), ((), ())),
                           preferred_element_type=jnp.float32)


def _leaky(h):
    return jnp.maximum(h, h * LEAK)


def _eye(n):
    r = lax.broadcasted_iota(jnp.int32, (n, n), 0)
    c = lax.broadcasted_iota(jnp.int32, (n, n), 1)
    return jnp.where(r == c, 1.0, 0.0)


def _outer(row, width):
    ones = jnp.ones((1, width), jnp.float32)
    return lax.dot_general(row, ones, (((0,), (0,)), ((), ())),
                           preferred_element_type=jnp.float32)


def _fused_kernel(e_hbm, idx_ref, xg_ref, gx_ref, bx_ref, ge_ref, be_ref,
                  w1_ref, b1_ref, w2_ref, b2_ref, w3_ref, b3_ref,
                  w4_ref, b4_ref, w5_ref, b5_ref, w6_ref, b6_ref,
                  out_ref,
                  e_scr, sums_ref, agt_ref, bgt_ref, w1ee_ref, b1cb_ref,
                  w2t_ref, b2cb_ref, w3t_ref, b3cb_ref,
                  w4t_ref, b4cb_ref, w5t_ref, b5cb_ref,
                  w6t_ref, b6cb_ref, sems,
                  *, n_edges, n_nf, n_groups, n_tiles):
    t = pl.program_id(0)

    @pl.when(t == 0)
    def _():
        sums_ref[...] = jnp.zeros_like(sums_ref)
        for i in range(n_tiles):
            pltpu.make_async_copy(
                e_hbm.at[pl.ds(i * TILE_E, TILE_E), :],
                e_scr.at[pl.ds(i * TILE_E, TILE_E), :],
                sems.at[i]).start()

    @pl.when(t < n_tiles)
    def _():
        off = pl.multiple_of(t * TILE_E, TILE_E)
        pltpu.make_async_copy(
            e_hbm.at[pl.ds(off, TILE_E), :],
            e_scr.at[pl.ds(off, TILE_E), :],
            sems.at[t]).wait()
        e = e_scr[pl.ds(off, TILE_E), :]
        sums_ref[0:1, :] = sums_ref[0:1, :] + jnp.sum(e, 0, keepdims=True)
        sums_ref[1:2, :] = sums_ref[1:2, :] + jnp.sum(e * e, 0, keepdims=True)

    @pl.when(t == n_tiles - 1)
    def _():
        w1e = w1_ref[2 * n_nf:, :]
        s = sums_ref[...]
        inv_n = jnp.float32(1.0 / n_edges)
        mean_e = s[0:1, :] * inv_n
        var_e = s[1:2, :] * inv_n - mean_e * mean_e
        scale_e = ge_ref[...] * lax.rsqrt(var_e + BN_EPS)
        shift_e = be_ref[...] - mean_e * scale_e
        w1et = _dot_ta_tb(w1e, _eye(n_nf))
        w1ee_ref[...] = (w1et * scale_e).astype(BF)
        b1_eff = b1_ref[...] + _dot_tb(shift_e, w1et)
        b1cb_ref[...] = _outer(b1_eff, HALF_E)

        xg = xg_ref[...]
        s4 = jnp.mean(xg, axis=0, keepdims=True)
        ss4 = jnp.mean(xg * xg, axis=0, keepdims=True)
        m = jnp.zeros((1, n_nf), jnp.float32)
        msq = jnp.zeros((1, n_nf), jnp.float32)
        for k in range(GROUP):
            m = m + s4[:, k * n_nf:(k + 1) * n_nf]
            msq = msq + ss4[:, k * n_nf:(k + 1) * n_nf]
        m = m * (1.0 / GROUP)
        msq = msq * (1.0 / GROUP)
        var_n = msq - m * m
        scale_n = gx_ref[...] * lax.rsqrt(var_n + BN_EPS)
        shift_n = bx_ref[...] - m * scale_n
        scale4 = jnp.concatenate([scale_n] * GROUP, axis=1)
        shift4 = jnp.concatenate([shift_n] * GROUP, axis=1)
        xn = xg * scale4 + shift4
        w1s = w1_ref[0:n_nf, :]
        w1d = w1_ref[n_nf:2 * n_nf, :]
        for k in range(GROUP):
            xk = xn[:, k * n_nf:(k + 1) * n_nf]
            agt_ref[k * 64:(k + 1) * 64, :] = _dot_ta_tb(w1s, xk).astype(BF)
            bgt_ref[k * 64:(k + 1) * 64, :] = _dot_ta_tb(w1d, xk).astype(BF)

        for w_ref, b_ref, wt_ref, bcb_ref in (
                (w2_ref, b2_ref, w2t_ref, b2cb_ref),
                (w3_ref, b3_ref, w3t_ref, b3cb_ref),
                (w4_ref, b4_ref, w4t_ref, b4cb_ref),
                (w5_ref, b5_ref, w5t_ref, b5cb_ref)):
            w = w_ref[...]
            wt_ref[...] = _dot_tb(_eye(w.shape[1]), w).astype(BF)
            bcb_ref[...] = _outer(b_ref[...], HALF_E)
        w6t = _dot_tb(_eye(2), w6_ref[...])
        w6t_ref[...] = jnp.concatenate(
            [w6t, jnp.zeros((6, 8), jnp.float32)], axis=0).astype(BF)
        b6cb_ref[...] = _outer(
            jnp.concatenate([b6_ref[...], jnp.zeros((1, 6), jnp.float32)],
                            axis=1), HALF_E)

    @pl.when(t >= n_tiles)
    def _():
        tb = t - n_tiles
        base = pl.multiple_of(tb * TILE_E, TILE_E)
        agt = agt_ref[...]
        bgt = bgt_ref[...]
        w1ee = w1ee_ref[...]
        b1cb = b1cb_ref[...]
        iota_g = lax.broadcasted_iota(jnp.int32, (n_groups, HALF_E), 0)
        n_half = TILE_E // HALF_E
        slices = [slice(h * HALF_E, (h + 1) * HALF_E) for h in range(n_half)]

        hs = []
        for h, sl in enumerate(slices):
            src = idx_ref[0:1, sl]
            dst = idx_ref[1:2, sl]
            ms = jnp.where(iota_g == (src >> 2), 1.0, 0.0).astype(BF)
            md = jnp.where(iota_g == (dst >> 2), 1.0, 0.0).astype(BF)
            gs = _dot(agt, ms)
            gd = _dot(bgt, md)
            srcr = src & 3
            dstr = dst & 3
            e_blk = e_scr[pl.ds(base + h * HALF_E, HALF_E), :]
            h1 = b1cb + _dot_tb(w1ee, e_blk.astype(BF))
            for k in range(GROUP):
                fs = jnp.where(srcr == k, 1.0, 0.0)
                fd = jnp.where(dstr == k, 1.0, 0.0)
                h1 = h1 + fs * gs[k * 64:(k + 1) * 64, :]
                h1 = h1 + fd * gd[k * 64:(k + 1) * 64, :]
            hs.append(_leaky(h1).astype(BF))
        for wt_ref, bcb_ref in ((w2t_ref, b2cb_ref), (w3t_ref, b3cb_ref),
                                (w4t_ref, b4cb_ref), (w5t_ref, b5cb_ref)):
            wt = wt_ref[...]
            bcb = bcb_ref[...]
            hs = [_leaky(_dot(wt, hh) + bcb).astype(BF) for hh in hs]
        w6t = w6t_ref[...]
        b6cb = b6cb_ref[...]
        for hh, sl in zip(hs, slices):
            out8 = _dot(w6t, hh) + b6cb
            out_ref[:, sl] = out8[0:2, :]


def kernel(x, edge_index, e, xbatch,
           bn_node_gamma, bn_node_beta, bn_edge_gamma, bn_edge_beta,
           w1, b1, w2, b2, w3, b3, w4, b4, w5, b5, w6, b6):
    del xbatch
    f32 = jnp.float32
    n_nodes, n_nf = x.shape
    n_edges, n_ef = e.shape
    n_groups = n_nodes // GROUP

    e_rows = _round_up(max(n_edges, 1), TILE_E)
    n_tiles = e_rows // TILE_E
    e_pad = e.astype(f32)
    idx_pad = edge_index.astype(jnp.int32)
    if e_rows != n_edges:
        e_pad = jnp.pad(e_pad, ((0, e_rows - n_edges), (0, 0)))
        idx_pad = jnp.pad(idx_pad, ((0, 0), (0, e_rows - n_edges)))
    xg = x.astype(f32).reshape(n_groups, GROUP * n_nf)

    smalls = [xg,
              bn_node_gamma.reshape(1, -1).astype(f32),
              bn_node_beta.reshape(1, -1).astype(f32),
              bn_edge_gamma.reshape(1, -1).astype(f32),
              bn_edge_beta.reshape(1, -1).astype(f32),
              w1.astype(f32), b1.reshape(1, -1).astype(f32),
              w2.astype(f32), b2.reshape(1, -1).astype(f32),
              w3.astype(f32), b3.reshape(1, -1).astype(f32),
              w4.astype(f32), b4.reshape(1, -1).astype(f32),
              w5.astype(f32), b5.reshape(1, -1).astype(f32),
              w6.astype(f32), b6.reshape(1, -1).astype(f32)]

    def _idx_map(t):
        return (0, jnp.maximum(t - n_tiles, 0))

    in_specs = (
        [pl.BlockSpec(memory_space=pl.ANY),
         pl.BlockSpec((2, TILE_E), _idx_map)]
        + [pl.BlockSpec(a.shape, lambda t, n=len(a.shape): (0,) * n)
           for a in smalls]
    )
    scratch = [
        pltpu.VMEM((e_rows, n_ef), f32),
        pltpu.VMEM((2, n_ef), f32),
        pltpu.VMEM((GROUP * 64, n_groups), BF),
        pltpu.VMEM((GROUP * 64, n_groups), BF),
        pltpu.VMEM((64, n_ef), BF),
        pltpu.VMEM((64, HALF_E), f32),
        pltpu.VMEM((64, 64), BF), pltpu.VMEM((64, HALF_E), f32),
        pltpu.VMEM((32, 64), BF), pltpu.VMEM((32, HALF_E), f32),
        pltpu.VMEM((16, 32), BF), pltpu.VMEM((16, HALF_E), f32),
        pltpu.VMEM((8, 16), BF), pltpu.VMEM((8, HALF_E), f32),
        pltpu.VMEM((8, 8), BF), pltpu.VMEM((8, HALF_E), f32),
        pltpu.SemaphoreType.DMA((n_tiles,)),
    ]
    mlp_flops = 2 * (64 * (2 * GROUP * 64 + n_ef) + 64 * 64 + 32 * 64
                     + 16 * 32 + 8 * 16 + 8 * 8)
    cost = pl.CostEstimate(
        flops=e_rows * mlp_flops + 2 * e_rows * n_groups * GROUP * 64,
        transcendentals=0,
        bytes_accessed=4 * e_rows * (n_ef + 4),
    )
    out2 = pl.pallas_call(
        functools.partial(_fused_kernel, n_edges=n_edges, n_nf=n_nf,
                          n_groups=n_groups, n_tiles=n_tiles),
        grid=(2 * n_tiles,),
        in_specs=in_specs,
        out_specs=pl.BlockSpec((2, TILE_E), _idx_map),
        out_shape=jax.ShapeDtypeStruct((2, e_rows), f32),
        scratch_shapes=scratch,
        compiler_params=pltpu.CompilerParams(
            dimension_semantics=("arbitrary",)),
        cost_estimate=cost,
    )(e_pad, idx_pad, *smalls)
    return {'edge_pred': [out2[:, :n_edges].T]}


# speedup vs baseline: 1.2927x; 1.1881x over previous
import functools
import jax
import jax.numpy as jnp
from jax import lax
from jax.experimental import pallas as pl
from jax.experimental.pallas import tpu as pltpu

LEAK = 0.1
BN_EPS = 1e-5
TILE_E = 16384
HALF_E = 1024
GROUP = 4
BF = jnp.bfloat16


def _round_up(a, b):
    return (a + b - 1) // b * b


def _dot(a, b):
    return jnp.dot(a, b, preferred_element_type=jnp.float32)


def _dot_tb(a, b):
    return lax.dot_general(a, b, (((1,), (1,)), ((), ())),
                           preferred_element_type=jnp.float32)


def _dot_ta_tb(a, b):
    return lax.dot_general(a, b, (((0,), (1,)), ((), ())),
                           preferred_element_type=jnp.float32)


def _leaky(h):
    return jnp.maximum(h, h * LEAK)


def _eye(n):
    r = lax.broadcasted_iota(jnp.int32, (n, n), 0)
    c = lax.broadcasted_iota(jnp.int32, (n, n), 1)
    return jnp.where(r == c, 1.0, 0.0)


def _outer(row, width):
    ones = jnp.ones((1, width), jnp.float32)
    return lax.dot_general(row, ones, (((0,), (0,)), ((), ())),
                           preferred_element_type=jnp.float32)


def _fused_kernel(e_hbm, idx_ref, xg_ref, gx_ref, bx_ref, ge_ref, be_ref,
                  w1_ref, b1_ref, w2_ref, b2_ref, w3_ref, b3_ref,
                  w4_ref, b4_ref, w5_ref, b5_ref, w6_ref, b6_ref,
                  out_ref,
                  e_scr, hnode_scr, sums_ref, agt_ref, bgt_ref, w1ee_ref,
                  b1cb_ref,
                  w2t_ref, b2cb_ref, w3t_ref, b3cb_ref,
                  w4t_ref, b4cb_ref, w5t_ref, b5cb_ref,
                  w6t_ref, b6cb_ref, sems,
                  *, n_edges, n_nf, n_groups, n_tiles):
    t = pl.program_id(0)

    @pl.when(t == 0)
    def _():
        sums_ref[...] = jnp.zeros_like(sums_ref)
        for i in range(n_tiles):
            pltpu.make_async_copy(
                e_hbm.at[pl.ds(i * TILE_E, TILE_E), :],
                e_scr.at[pl.ds(i * TILE_E, TILE_E), :],
                sems.at[i]).start()
        xg = xg_ref[...]
        s4 = jnp.mean(xg, axis=0, keepdims=True)
        ss4 = jnp.mean(xg * xg, axis=0, keepdims=True)
        m = jnp.zeros((1, n_nf), jnp.float32)
        msq = jnp.zeros((1, n_nf), jnp.float32)
        for k in range(GROUP):
            m = m + s4[:, k * n_nf:(k + 1) * n_nf]
            msq = msq + ss4[:, k * n_nf:(k + 1) * n_nf]
        m = m * (1.0 / GROUP)
        msq = msq * (1.0 / GROUP)
        var_n = msq - m * m
        scale_n = gx_ref[...] * lax.rsqrt(var_n + BN_EPS)
        shift_n = bx_ref[...] - m * scale_n
        scale4 = jnp.concatenate([scale_n] * GROUP, axis=1)
        shift4 = jnp.concatenate([shift_n] * GROUP, axis=1)
        xn = xg * scale4 + shift4
        w1s = w1_ref[0:n_nf, :]
        w1d = w1_ref[n_nf:2 * n_nf, :]
        for k in range(GROUP):
            xk = xn[:, k * n_nf:(k + 1) * n_nf]
            agt_ref[k * 64:(k + 1) * 64, :] = _dot_ta_tb(w1s, xk).astype(BF)
            bgt_ref[k * 64:(k + 1) * 64, :] = _dot_ta_tb(w1d, xk).astype(BF)

    @pl.when(t < n_tiles)
    def _():
        off = pl.multiple_of(t * TILE_E, TILE_E)
        pltpu.make_async_copy(
            e_hbm.at[pl.ds(off, TILE_E), :],
            e_scr.at[pl.ds(off, TILE_E), :],
            sems.at[t]).wait()
        e = e_scr[pl.ds(off, TILE_E), :]
        sums_ref[0:1, :] = sums_ref[0:1, :] + jnp.sum(e, 0, keepdims=True)
        sums_ref[1:2, :] = sums_ref[1:2, :] + jnp.sum(e * e, 0, keepdims=True)
        agt = agt_ref[...]
        bgt = bgt_ref[...]
        iota_g = lax.broadcasted_iota(jnp.int32, (n_groups, HALF_E), 0)
        for h in range(TILE_E // HALF_E):
            sl = slice(h * HALF_E, (h + 1) * HALF_E)
            src = idx_ref[0:1, sl]
            dst = idx_ref[1:2, sl]
            ms = jnp.where(iota_g == (src >> 2), 1.0, 0.0).astype(BF)
            md = jnp.where(iota_g == (dst >> 2), 1.0, 0.0).astype(BF)
            gs = _dot(agt, ms)
            gd = _dot(bgt, md)
            srcr = src & 3
            dstr = dst & 3
            hn = jnp.zeros((64, HALF_E), jnp.float32)
            for k in range(GROUP):
                fs = jnp.where(srcr == k, 1.0, 0.0)
                fd = jnp.where(dstr == k, 1.0, 0.0)
                hn = hn + fs * gs[k * 64:(k + 1) * 64, :]
                hn = hn + fd * gd[k * 64:(k + 1) * 64, :]
            hnode_scr[t, :, sl] = hn.astype(BF)

    @pl.when(t == n_tiles - 1)
    def _():
        w1e = w1_ref[2 * n_nf:, :]
        s = sums_ref[...]
        inv_n = jnp.float32(1.0 / n_edges)
        mean_e = s[0:1, :] * inv_n
        var_e = s[1:2, :] * inv_n - mean_e * mean_e
        scale_e = ge_ref[...] * lax.rsqrt(var_e + BN_EPS)
        shift_e = be_ref[...] - mean_e * scale_e
        w1et = _dot_ta_tb(w1e, _eye(n_nf))
        w1ee_ref[...] = (w1et * scale_e).astype(BF)
        b1_eff = b1_ref[...] + _dot_tb(shift_e, w1et)
        b1cb_ref[...] = _outer(b1_eff, HALF_E)

        for w_ref, b_ref, wt_ref, bcb_ref in (
                (w2_ref, b2_ref, w2t_ref, b2cb_ref),
                (w3_ref, b3_ref, w3t_ref, b3cb_ref),
                (w4_ref, b4_ref, w4t_ref, b4cb_ref),
                (w5_ref, b5_ref, w5t_ref, b5cb_ref)):
            w = w_ref[...]
            wt_ref[...] = _dot_tb(_eye(w.shape[1]), w).astype(BF)
            bcb_ref[...] = _outer(b_ref[...], HALF_E)
        w6t = _dot_tb(_eye(2), w6_ref[...])
        w6t_ref[...] = jnp.concatenate(
            [w6t, jnp.zeros((6, 8), jnp.float32)], axis=0).astype(BF)
        b6cb_ref[...] = _outer(
            jnp.concatenate([b6_ref[...], jnp.zeros((1, 6), jnp.float32)],
                            axis=1), HALF_E)

    @pl.when(t >= n_tiles)
    def _():
        tb = t - n_tiles
        base = pl.multiple_of(tb * TILE_E, TILE_E)
        w1ee = w1ee_ref[...]
        b1cb = b1cb_ref[...]
        n_half = TILE_E // HALF_E
        slices = [slice(h * HALF_E, (h + 1) * HALF_E) for h in range(n_half)]

        hs = []
        for h, sl in enumerate(slices):
            e_blk = e_scr[pl.ds(base + h * HALF_E, HALF_E), :]
            h1 = b1cb + _dot_tb(w1ee, e_blk.astype(BF))
            h1 = h1 + hnode_scr[tb, :, sl].astype(jnp.float32)
            hs.append(_leaky(h1).astype(BF))
        for wt_ref, bcb_ref in ((w2t_ref, b2cb_ref), (w3t_ref, b3cb_ref),
                                (w4t_ref, b4cb_ref), (w5t_ref, b5cb_ref)):
            wt = wt_ref[...]
            bcb = bcb_ref[...]
            hs = [_leaky(_dot(wt, hh) + bcb).astype(BF) for hh in hs]
        w6t = w6t_ref[...]
        b6cb = b6cb_ref[...]
        for hh, sl in zip(hs, slices):
            out8 = _dot(w6t, hh) + b6cb
            out_ref[:, sl] = out8[0:2, :]


def kernel(x, edge_index, e, xbatch,
           bn_node_gamma, bn_node_beta, bn_edge_gamma, bn_edge_beta,
           w1, b1, w2, b2, w3, b3, w4, b4, w5, b5, w6, b6):
    del xbatch
    f32 = jnp.float32
    n_nodes, n_nf = x.shape
    n_edges, n_ef = e.shape
    n_groups = n_nodes // GROUP

    e_rows = _round_up(max(n_edges, 1), TILE_E)
    n_tiles = e_rows // TILE_E
    e_pad = e.astype(f32)
    idx_pad = edge_index.astype(jnp.int32)
    if e_rows != n_edges:
        e_pad = jnp.pad(e_pad, ((0, e_rows - n_edges), (0, 0)))
        idx_pad = jnp.pad(idx_pad, ((0, 0), (0, e_rows - n_edges)))
    xg = x.astype(f32).reshape(n_groups, GROUP * n_nf)

    smalls = [xg,
              bn_node_gamma.reshape(1, -1).astype(f32),
              bn_node_beta.reshape(1, -1).astype(f32),
              bn_edge_gamma.reshape(1, -1).astype(f32),
              bn_edge_beta.reshape(1, -1).astype(f32),
              w1.astype(f32), b1.reshape(1, -1).astype(f32),
              w2.astype(f32), b2.reshape(1, -1).astype(f32),
              w3.astype(f32), b3.reshape(1, -1).astype(f32),
              w4.astype(f32), b4.reshape(1, -1).astype(f32),
              w5.astype(f32), b5.reshape(1, -1).astype(f32),
              w6.astype(f32), b6.reshape(1, -1).astype(f32)]

    def _idx_map(t):
        return (0, lax.rem(t, n_tiles))

    in_specs = (
        [pl.BlockSpec(memory_space=pl.ANY),
         pl.BlockSpec((2, TILE_E), _idx_map)]
        + [pl.BlockSpec(a.shape, lambda t, n=len(a.shape): (0,) * n)
           for a in smalls]
    )
    scratch = [
        pltpu.VMEM((e_rows, n_ef), f32),
        pltpu.VMEM((n_tiles, 64, TILE_E), BF),
        pltpu.VMEM((2, n_ef), f32),
        pltpu.VMEM((GROUP * 64, n_groups), BF),
        pltpu.VMEM((GROUP * 64, n_groups), BF),
        pltpu.VMEM((64, n_ef), BF),
        pltpu.VMEM((64, HALF_E), f32),
        pltpu.VMEM((64, 64), BF), pltpu.VMEM((64, HALF_E), f32),
        pltpu.VMEM((32, 64), BF), pltpu.VMEM((32, HALF_E), f32),
        pltpu.VMEM((16, 32), BF), pltpu.VMEM((16, HALF_E), f32),
        pltpu.VMEM((8, 16), BF), pltpu.VMEM((8, HALF_E), f32),
        pltpu.VMEM((8, 8), BF), pltpu.VMEM((8, HALF_E), f32),
        pltpu.SemaphoreType.DMA((n_tiles,)),
    ]
    mlp_flops = 2 * (64 * (2 * GROUP * 64 + n_ef) + 64 * 64 + 32 * 64
                     + 16 * 32 + 8 * 16 + 8 * 8)
    cost = pl.CostEstimate(
        flops=e_rows * mlp_flops + 2 * e_rows * n_groups * GROUP * 64,
        transcendentals=0,
        bytes_accessed=4 * e_rows * (n_ef + 4),
    )
    out2 = pl.pallas_call(
        functools.partial(_fused_kernel, n_edges=n_edges, n_nf=n_nf,
                          n_groups=n_groups, n_tiles=n_tiles),
        grid=(2 * n_tiles,),
        in_specs=in_specs,
        out_specs=pl.BlockSpec((2, TILE_E), _idx_map),
        out_shape=jax.ShapeDtypeStruct((2, e_rows), f32),
        scratch_shapes=scratch,
        compiler_params=pltpu.CompilerParams(
            dimension_semantics=("arbitrary",)),
        cost_estimate=cost,
    )(e_pad, idx_pad, *smalls)
    return {'edge_pred': [out2[:, :n_edges].T]}
